# baseline (speedup 1.0000x reference)
import jax
import jax.numpy as jnp
from jax import lax
from jax.experimental import pallas as pl
from jax.experimental.pallas import tpu as pltpu
from jax.experimental.pallas import tpu_sc as plsc

B = 16384
DIM = 32
VOCAB = 1000000
NC = 2
NS = 16
NW = NC * NS
BPW = B // NW
L = 16
RP = 64
NROUND = BPW // RP
GPR = RP // L


def _body(pairs_hbm, emb_hbm, ctx_hbm, out_hbm,
          pv, erows, crows, outv, sems):
    wid = lax.axis_index("s") * NC + lax.axis_index("c")
    base = wid * BPW

    pltpu.sync_copy(pairs_hbm.at[pl.ds(2 * base, 2 * BPW)], pv)

    iota = lax.iota(jnp.int32, L)
    dcols = [jnp.full((L,), d, jnp.int32) for d in range(DIM)]

    def fire(g, _, slot):
        ids0 = pv[pl.ds(2 * L * g, L)]
        ids1 = pv[pl.ds(2 * L * g + L, L)]
        for lane in range(L):
            ids = ids0 if lane < L // 2 else ids1
            c = ids[(2 * lane) % L]
            t = ids[(2 * lane + 1) % L]
            row = (g % GPR) * L + lane
            pltpu.async_copy(
                emb_hbm.at[pl.ds(c, 1), :],
                erows.at[slot, pl.ds(row, 1), :], sems.at[slot])
            pltpu.async_copy(
                ctx_hbm.at[pl.ds(t, 1), :],
                crows.at[slot, pl.ds(row, 1), :], sems.at[slot])
        return 0

    def drain(r):
        slot = r % 2
        pltpu.make_async_copy(
            emb_hbm.at[pl.ds(0, RP), :], erows.at[slot], sems.at[slot]).wait()
        pltpu.make_async_copy(
            ctx_hbm.at[pl.ds(0, RP), :], crows.at[slot], sems.at[slot]).wait()

    def group(g, _, slot):
        lrow = (g % GPR) * L + iota
        acc = jnp.zeros((L,), jnp.float32)
        for d in range(DIM):
            a = plsc.load_gather(erows.at[slot], [lrow, dcols[d]])
            b = plsc.load_gather(crows.at[slot], [lrow, dcols[d]])
            acc = acc + a * b
        y = 1.0 / (1.0 + jnp.exp(-acc))
        plsc.store_scatter(outv, [g * L + iota], y)
        return 0

    import functools as _ft
    for r in range(NROUND):
        lax.fori_loop(r * GPR, (r + 1) * GPR,
                      _ft.partial(fire, slot=r % 2), 0)
        if r > 0:
            drain(r - 1)
            lax.fori_loop((r - 1) * GPR, r * GPR,
                          _ft.partial(group, slot=(r - 1) % 2), 0)
    drain(NROUND - 1)
    lax.fori_loop((NROUND - 1) * GPR, NROUND * GPR,
                  _ft.partial(group, slot=(NROUND - 1) % 2), 0)

    pltpu.sync_copy(outv, out_hbm.at[pl.ds(base, BPW)])


@jax.jit
def _skipgram(pairs, emb_w, ctx_w):
    mesh = plsc.VectorSubcoreMesh(core_axis_name="c", subcore_axis_name="s")
    k = pl.kernel(
        _body,
        out_type=jax.ShapeDtypeStruct((B,), jnp.float32),
        mesh=mesh,
        compiler_params=pltpu.CompilerParams(
            needs_layout_passes=False, skip_device_barrier=True),
        scratch_types=[
            pltpu.VMEM((2 * BPW,), jnp.int32),
            pltpu.VMEM((2, RP, DIM), jnp.float32),
            pltpu.VMEM((2, RP, DIM), jnp.float32),
            pltpu.VMEM((BPW,), jnp.float32),
            pltpu.SemaphoreType.DMA((2,)),
        ],
    )
    return k(pairs, emb_w, ctx_w)


def kernel(pairs, emb_w, ctx_w):
    return _skipgram(pairs.astype(jnp.int32).reshape(-1), emb_w, ctx_w)


# speedup vs baseline: 1.0008x; 1.0008x over previous
import jax
import jax.numpy as jnp
from jax import lax
from jax.experimental import pallas as pl
from jax.experimental.pallas import tpu as pltpu
from jax.experimental.pallas import tpu_sc as plsc

B = 16384
DIM = 32
VOCAB = 1000000
NC = 2
NS = 16
NW = NC * NS
BPW = B // NW
L = 16
RP = 64
NROUND = BPW // RP
GPR = RP // L


def _body(pairs_hbm, emb_hbm, ctx_hbm, out_hbm,
          pv, erows, crows, outv, sems):
    wid = lax.axis_index("s") * NC + lax.axis_index("c")
    base = wid * BPW

    pltpu.sync_copy(pairs_hbm.at[pl.ds(2 * base, 2 * BPW)], pv)

    iota = lax.iota(jnp.int32, L)
    dcols = [jnp.full((L,), d, jnp.int32) for d in range(DIM)]

    def fire(g, _, slot):
        ids0 = pv[pl.ds(2 * L * g, L)]
        ids1 = pv[pl.ds(2 * L * g + L, L)]
        for lane in range(L):
            ids = ids0 if lane < L // 2 else ids1
            c = ids[(2 * lane) % L]
            t = ids[(2 * lane + 1) % L]
            row = (g % GPR) * L + lane
            pltpu.async_copy(
                emb_hbm.at[pl.ds(c, 1), :],
                erows.at[slot, pl.ds(row, 1), :], sems.at[slot])
            pltpu.async_copy(
                ctx_hbm.at[pl.ds(t, 1), :],
                crows.at[slot, pl.ds(row, 1), :], sems.at[slot])
        return 0

    def drain(r):
        slot = r % 2
        pltpu.make_async_copy(
            emb_hbm.at[pl.ds(0, RP), :], erows.at[slot], sems.at[slot]).wait()
        pltpu.make_async_copy(
            ctx_hbm.at[pl.ds(0, RP), :], crows.at[slot], sems.at[slot]).wait()

    def group(g, _, slot):
        lrow = (g % GPR) * L + iota
        acc = jnp.zeros((L,), jnp.float32)
        for d in range(DIM):
            a = plsc.load_gather(erows.at[slot], [lrow, dcols[d]])
            b = plsc.load_gather(crows.at[slot], [lrow, dcols[d]])
            acc = acc + a * b
        y = 1.0 / (1.0 + jnp.exp(-acc))
        plsc.store_scatter(outv, [g * L + iota], y)
        return 0

    import functools as _ft
    for r in range(NROUND):
        lax.fori_loop(r * GPR, (r + 1) * GPR,
                      _ft.partial(fire, slot=r % 2), 0)
        if r > 0:
            drain(r - 1)
            lax.fori_loop((r - 1) * GPR, r * GPR,
                          _ft.partial(group, slot=(r - 1) % 2), 0)
    drain(NROUND - 1)
    lax.fori_loop((NROUND - 1) * GPR, NROUND * GPR,
                  _ft.partial(group, slot=(NROUND - 1) % 2), 0)

    pltpu.sync_copy(outv, out_hbm.at[pl.ds(base, BPW)])


@jax.jit
def _skipgram(pairs, emb_w, ctx_w):
    mesh = plsc.VectorSubcoreMesh(core_axis_name="c", subcore_axis_name="s")
    k = pl.kernel(
        _body,
        out_type=jax.ShapeDtypeStruct((B,), jnp.float32),
        mesh=mesh,
        compiler_params=pltpu.CompilerParams(needs_layout_passes=False),
        scratch_types=[
            pltpu.VMEM((2 * BPW,), jnp.int32),
            pltpu.VMEM((2, RP, DIM), jnp.float32),
            pltpu.VMEM((2, RP, DIM), jnp.float32),
            pltpu.VMEM((BPW,), jnp.float32),
            pltpu.SemaphoreType.DMA((2,)),
        ],
    )
    return k(pairs, emb_w, ctx_w)


def kernel(pairs, emb_w, ctx_w):
    return _skipgram(pairs.astype(jnp.int32).reshape(-1), emb_w, ctx_w)


# speedup vs baseline: 1.0019x; 1.0011x over previous
import jax
import jax.numpy as jnp
from jax import lax
from jax.experimental import pallas as pl
from jax.experimental.pallas import tpu as pltpu
from jax.experimental.pallas import tpu_sc as plsc

B = 16384
DIM = 32
VOCAB = 1000000
NC = 2
NS = 16
NW = NC * NS
BPW = B // NW
L = 16
RP = 64
NROUND = BPW // RP
GPR = RP // L
NQ = 4


def _body(pairs_hbm, emb_hbm, ctx_hbm, out_hbm,
          pv, erows, crows, outv, sems):
    wid = lax.axis_index("s") * NC + lax.axis_index("c")
    base = wid * BPW

    pltpu.sync_copy(pairs_hbm.at[pl.ds(2 * base, 2 * BPW)], pv)

    iota = lax.iota(jnp.int32, L)
    dcols = [jnp.full((L,), d, jnp.int32) for d in range(DIM)]

    def fire(g, _, slot):
        ids0 = pv[pl.ds(2 * L * g, L)]
        ids1 = pv[pl.ds(2 * L * g + L, L)]
        for lane in range(L):
            ids = ids0 if lane < L // 2 else ids1
            c = ids[(2 * lane) % L]
            t = ids[(2 * lane + 1) % L]
            row = (g % GPR) * L + lane
            pltpu.async_copy(
                emb_hbm.at[pl.ds(c, 1), :],
                erows.at[slot, pl.ds(row, 1), :], sems.at[slot, lane % NQ])
            pltpu.async_copy(
                ctx_hbm.at[pl.ds(t, 1), :],
                crows.at[slot, pl.ds(row, 1), :], sems.at[slot, lane % NQ])
        return 0

    def drain(r):
        slot = r % 2
        for q in range(NQ):
            pltpu.make_async_copy(
                emb_hbm.at[pl.ds(0, RP // NQ), :],
                erows.at[slot, pl.ds(0, RP // NQ), :],
                sems.at[slot, q]).wait()
            pltpu.make_async_copy(
                ctx_hbm.at[pl.ds(0, RP // NQ), :],
                crows.at[slot, pl.ds(0, RP // NQ), :],
                sems.at[slot, q]).wait()

    def group(g, _, slot):
        lrow = (g % GPR) * L + iota
        acc = jnp.zeros((L,), jnp.float32)
        for d in range(DIM):
            a = plsc.load_gather(erows.at[slot], [lrow, dcols[d]])
            b = plsc.load_gather(crows.at[slot], [lrow, dcols[d]])
            acc = acc + a * b
        y = 1.0 / (1.0 + jnp.exp(-acc))
        plsc.store_scatter(outv, [g * L + iota], y)
        return 0

    import functools as _ft
    for r in range(NROUND):
        lax.fori_loop(r * GPR, (r + 1) * GPR,
                      _ft.partial(fire, slot=r % 2), 0)
        if r > 0:
            drain(r - 1)
            lax.fori_loop((r - 1) * GPR, r * GPR,
                          _ft.partial(group, slot=(r - 1) % 2), 0)
    drain(NROUND - 1)
    lax.fori_loop((NROUND - 1) * GPR, NROUND * GPR,
                  _ft.partial(group, slot=(NROUND - 1) % 2), 0)

    pltpu.sync_copy(outv, out_hbm.at[pl.ds(base, BPW)])


@jax.jit
def _skipgram(pairs, emb_w, ctx_w):
    mesh = plsc.VectorSubcoreMesh(core_axis_name="c", subcore_axis_name="s")
    k = pl.kernel(
        _body,
        out_type=jax.ShapeDtypeStruct((B,), jnp.float32),
        mesh=mesh,
        compiler_params=pltpu.CompilerParams(needs_layout_passes=False),
        scratch_types=[
            pltpu.VMEM((2 * BPW,), jnp.int32),
            pltpu.VMEM((2, RP, DIM), jnp.float32),
            pltpu.VMEM((2, RP, DIM), jnp.float32),
            pltpu.VMEM((BPW,), jnp.float32),
            pltpu.SemaphoreType.DMA((2, NQ)),
        ],
    )
    return k(pairs, emb_w, ctx_w)


def kernel(pairs, emb_w, ctx_w):
    return _skipgram(pairs.astype(jnp.int32).reshape(-1), emb_w, ctx_w)
